# baseline (speedup 1.0000x reference)
import jax
import jax.numpy as jnp
from jax.experimental import pallas as pl
from jax.experimental.pallas import tpu as pltpu

BATCH = 16
N = 512
EMB_DIM = 128
FEAT_DIM = 32
HEADS = 4
ROWS = 64


def _gat_kernel(x_ref, adj_ref, w_ref, b_ref, wsrc_ref, wdst_ref,
                wh_ref, bh_ref, attn_ref, out_ref,
                h_scr, s_scr, d_scr, fo_scr):
    x = x_ref[0]
    for hi in range(HEADS):
        h = jnp.dot(x, w_ref[hi], preferred_element_type=jnp.float32)
        h_scr[hi] = h
        th = jnp.tanh(h)
        s_scr[hi] = jnp.sum(th * wsrc_ref[0, hi], axis=1, keepdims=True)
        d_scr[hi] = jnp.sum(th * wdst_ref[0, hi], axis=1, keepdims=True).T

    neg = jnp.float32(-1e12)

    def tile_body(t, carry):
        r0 = t * ROWS
        adj_t = adj_ref[0, pl.ds(r0, ROWS), :]
        mask = adj_t > 0
        for hi in range(HEADS):
            s_t = s_scr[hi, pl.ds(r0, ROWS), :]
            z = s_t + d_scr[hi]
            z = jnp.maximum(z, 0.2 * z)
            z = jnp.where(mask, z, neg)
            m = jnp.max(z, axis=1, keepdims=True)
            e = jnp.exp(z - m)
            r = 1.0 / jnp.sum(e, axis=1, keepdims=True)
            p = e * r
            attn_ref[0, hi, pl.ds(r0, ROWS), :] = p
            fo = jnp.dot(p, h_scr[hi],
                         preferred_element_type=jnp.float32) + b_ref[0]
            fo_scr[hi, pl.ds(r0, ROWS), :] = jnp.where(
                fo > 0, fo, jnp.exp(jnp.minimum(fo, 0.0)) - 1.0)
        return carry

    jax.lax.fori_loop(0, N // ROWS, tile_body, 0, unroll=False)

    fo_cat = jnp.concatenate([fo_scr[hi] for hi in range(HEADS)], axis=1)
    gate = jax.nn.sigmoid(
        jnp.dot(x, wh_ref[...], preferred_element_type=jnp.float32)
        + bh_ref[0])
    out_ref[0] = gate * fo_cat + (1.0 - gate) * x


def kernel(doc_sents_h, doc_len, adj, W, b, w_src, w_dst, Wh, bh):
    del doc_len
    b2 = b.reshape(1, FEAT_DIM)
    wsrc = w_src.reshape(1, HEADS, FEAT_DIM)
    wdst = w_dst.reshape(1, HEADS, FEAT_DIM)
    bh2 = bh.reshape(1, HEADS * FEAT_DIM)

    attn, feat_out = pl.pallas_call(
        _gat_kernel,
        grid=(BATCH,),
        in_specs=[
            pl.BlockSpec((1, N, EMB_DIM), lambda bi: (bi, 0, 0)),
            pl.BlockSpec((1, N, N), lambda bi: (bi, 0, 0)),
            pl.BlockSpec((HEADS, EMB_DIM, FEAT_DIM), lambda bi: (0, 0, 0)),
            pl.BlockSpec((1, FEAT_DIM), lambda bi: (0, 0)),
            pl.BlockSpec((1, HEADS, FEAT_DIM), lambda bi: (0, 0, 0)),
            pl.BlockSpec((1, HEADS, FEAT_DIM), lambda bi: (0, 0, 0)),
            pl.BlockSpec((EMB_DIM, HEADS * FEAT_DIM), lambda bi: (0, 0)),
            pl.BlockSpec((1, HEADS * FEAT_DIM), lambda bi: (0, 0)),
        ],
        out_specs=[
            pl.BlockSpec((1, HEADS, N, N), lambda bi: (bi, 0, 0, 0)),
            pl.BlockSpec((1, N, HEADS * FEAT_DIM), lambda bi: (bi, 0, 0)),
        ],
        out_shape=[
            jax.ShapeDtypeStruct((BATCH, HEADS, N, N), jnp.float32),
            jax.ShapeDtypeStruct((BATCH, N, HEADS * FEAT_DIM), jnp.float32),
        ],
        scratch_shapes=[
            pltpu.VMEM((HEADS, N, FEAT_DIM), jnp.float32),
            pltpu.VMEM((HEADS, N, 1), jnp.float32),
            pltpu.VMEM((HEADS, 1, N), jnp.float32),
            pltpu.VMEM((HEADS, N, FEAT_DIM), jnp.float32),
        ],
        compiler_params=pltpu.CompilerParams(
            dimension_semantics=("parallel",),
        ),
    )(doc_sents_h, adj, W, b2, wsrc, wdst, Wh, bh2)
    return feat_out, attn


# speedup vs baseline: 1.5774x; 1.5774x over previous
import jax
import jax.numpy as jnp
from jax.experimental import pallas as pl
from jax.experimental.pallas import tpu as pltpu

BATCH = 16
N = 512
EMB_DIM = 128
FEAT_DIM = 32
HEADS = 4


def _gat_kernel(x_ref, adj_ref, w_ref, b_ref, wsrc_ref, wdst_ref,
                wh_ref, bh_ref, attn_ref, out_ref):
    x = x_ref[0]
    mask = adj_ref[0] > 0
    neg = jnp.float32(-1e12)
    outs = []
    for hi in range(HEADS):
        h = jnp.dot(x, w_ref[hi], preferred_element_type=jnp.float32)
        th = jnp.tanh(h)
        s = jnp.sum(th * wsrc_ref[0, hi], axis=1, keepdims=True)
        d = jnp.sum(th * wdst_ref[0, hi], axis=1, keepdims=True)
        logits = s + d.T
        logits = jnp.maximum(logits, 0.2 * logits)
        logits = jnp.where(mask, logits, neg)
        m = jnp.max(logits, axis=1, keepdims=True)
        e = jnp.exp(logits - m)
        p = e * (1.0 / jnp.sum(e, axis=1, keepdims=True))
        attn_ref[0, hi] = p
        fo = jnp.dot(p, h, preferred_element_type=jnp.float32) + b_ref[0]
        outs.append(jnp.where(fo > 0, fo, jnp.exp(jnp.minimum(fo, 0.0)) - 1.0))
    fo_cat = jnp.concatenate(outs, axis=1)
    gate = jax.nn.sigmoid(
        jnp.dot(x, wh_ref[...], preferred_element_type=jnp.float32)
        + bh_ref[0])
    out_ref[0] = gate * fo_cat + (1.0 - gate) * x


def kernel(doc_sents_h, doc_len, adj, W, b, w_src, w_dst, Wh, bh):
    del doc_len
    b2 = b.reshape(1, FEAT_DIM)
    wsrc = w_src.reshape(1, HEADS, FEAT_DIM)
    wdst = w_dst.reshape(1, HEADS, FEAT_DIM)
    bh2 = bh.reshape(1, HEADS * FEAT_DIM)

    attn, feat_out = pl.pallas_call(
        _gat_kernel,
        grid=(BATCH,),
        in_specs=[
            pl.BlockSpec((1, N, EMB_DIM), lambda bi: (bi, 0, 0)),
            pl.BlockSpec((1, N, N), lambda bi: (bi, 0, 0)),
            pl.BlockSpec((HEADS, EMB_DIM, FEAT_DIM), lambda bi: (0, 0, 0)),
            pl.BlockSpec((1, FEAT_DIM), lambda bi: (0, 0)),
            pl.BlockSpec((1, HEADS, FEAT_DIM), lambda bi: (0, 0, 0)),
            pl.BlockSpec((1, HEADS, FEAT_DIM), lambda bi: (0, 0, 0)),
            pl.BlockSpec((EMB_DIM, HEADS * FEAT_DIM), lambda bi: (0, 0)),
            pl.BlockSpec((1, HEADS * FEAT_DIM), lambda bi: (0, 0)),
        ],
        out_specs=[
            pl.BlockSpec((1, HEADS, N, N), lambda bi: (bi, 0, 0, 0)),
            pl.BlockSpec((1, N, HEADS * FEAT_DIM), lambda bi: (bi, 0, 0)),
        ],
        out_shape=[
            jax.ShapeDtypeStruct((BATCH, HEADS, N, N), jnp.float32),
            jax.ShapeDtypeStruct((BATCH, N, HEADS * FEAT_DIM), jnp.float32),
        ],
        compiler_params=pltpu.CompilerParams(
            dimension_semantics=("parallel",),
        ),
    )(doc_sents_h, adj, W, b2, wsrc, wdst, Wh, bh2)
    return feat_out, attn
